# Initial kernel scaffold; baseline (speedup 1.0000x reference)
import jax
import jax.numpy as jnp
from jax.experimental import pallas as pl


def kernel(z, emb):
    raise NotImplementedError("write your pallas kernel here")


# speedup vs baseline: 1.7393x; 1.7393x over previous
import functools

import jax
import jax.numpy as jnp
from jax import lax
from jax.experimental import pallas as pl
from jax.experimental.pallas import tpu as pltpu
from jax.experimental.pallas import tpu_sc as plsc

VQ_C = 1.0
COMMIT_C = 0.5


def _tc_body(z_ref, emb_ref, amin_ref, lsum_ref):
    zt = z_ref[...]
    e = emb_ref[...]
    cross = lax.dot_general(zt, e, (((1,), (1,)), ((), ())),
                            preferred_element_type=jnp.float32)
    z2 = jnp.sum(zt * zt, axis=1, keepdims=True)
    e2 = jnp.sum(e * e, axis=1)
    dist = (z2 - 2.0 * cross) + e2[None, :]
    amin_ref[0, 0, :] = jnp.argmin(dist, axis=1).astype(jnp.int32)

    @pl.when(pl.program_id(0) == 0)
    def _():
        lsum_ref[0, 0] = 0.0

    lsum_ref[0, 0] += jnp.sum(jnp.min(dist, axis=1))


def _tc_argmin_loss(zf, emb, tile):
    n, d = zf.shape
    k = emb.shape[0]
    grid = n // tile
    return pl.pallas_call(
        _tc_body,
        grid=(grid,),
        in_specs=[
            pl.BlockSpec((tile, d), lambda i: (i, 0)),
            pl.BlockSpec((k, d), lambda i: (0, 0)),
        ],
        out_specs=[
            pl.BlockSpec((1, 1, tile), lambda i: (i, 0, 0)),
            pl.BlockSpec(memory_space=pltpu.SMEM),
        ],
        out_shape=[
            jax.ShapeDtypeStruct((grid, 1, tile), jnp.int32),
            jax.ShapeDtypeStruct((1, 1), jnp.float32),
        ],
    )(zf, emb)


def _sc_gather(emb, idx, n, d):
    info = plsc.get_sparse_core_info()
    nc, ns = info.num_cores, info.num_subcores
    nw = nc * ns
    b_per_w = n // nw
    ch = 128
    nch = b_per_w // ch
    mesh = plsc.VectorSubcoreMesh(core_axis_name="c", subcore_axis_name="s")

    @functools.partial(
        pl.kernel,
        mesh=mesh,
        out_type=jax.ShapeDtypeStruct((n, d), jnp.float32),
        scratch_types=[
            pltpu.VMEM((b_per_w,), jnp.int32),
            pltpu.VMEM((ch, d), jnp.float32),
            pltpu.VMEM((ch, d), jnp.float32),
            pltpu.SemaphoreType.DMA,
            pltpu.SemaphoreType.DMA,
        ],
    )
    def gather_kernel(emb_hbm, idx_hbm, out_hbm, idx_v, buf0, buf1, sem0, sem1):
        wid = lax.axis_index("s") * nc + lax.axis_index("c")
        base = wid * b_per_w
        pltpu.sync_copy(idx_hbm.at[pl.ds(base, b_per_w)], idx_v)
        bufs = (buf0, buf1)
        sems = (sem0, sem1)
        cps = [None, None]
        cps[0] = pltpu.async_copy(
            emb_hbm.at[idx_v.at[pl.ds(0, ch)]], bufs[0], sems[0])
        for c in range(nch):
            if c + 1 < nch:
                cps[(c + 1) % 2] = pltpu.async_copy(
                    emb_hbm.at[idx_v.at[pl.ds((c + 1) * ch, ch)]],
                    bufs[(c + 1) % 2], sems[(c + 1) % 2])
            cps[c % 2].wait()
            pltpu.sync_copy(bufs[c % 2], out_hbm.at[pl.ds(base + c * ch, ch)])

    return gather_kernel(emb, idx)


def kernel(z, emb):
    b, t, d = z.shape
    n = b * t
    zf = z.reshape(n, d)
    amin3, lsum = _tc_argmin_loss(zf, emb, tile=1024)
    amin_flat = amin3.reshape(n)
    z_q = _sc_gather(emb, amin_flat, n, d)
    loss = lsum[0, 0] * ((VQ_C + COMMIT_C) / (n * d))
    return z_q.reshape(b, t, d), amin_flat.reshape(b, t), loss


# speedup vs baseline: 1.9793x; 1.1380x over previous
import functools

import jax
import jax.numpy as jnp
from jax import lax
from jax.experimental import pallas as pl
from jax.experimental.pallas import tpu as pltpu
from jax.experimental.pallas import tpu_sc as plsc

VQ_C = 1.0
COMMIT_C = 0.5


def _tc_body(z_ref, emb_ref, amin_ref, lsum_ref):
    zt = z_ref[...]
    e = emb_ref[...]
    cross = lax.dot_general(zt, e, (((1,), (1,)), ((), ())),
                            preferred_element_type=jnp.float32)
    z2 = jnp.sum(zt * zt, axis=1, keepdims=True)
    e2 = jnp.sum(e * e, axis=1)
    dist = (z2 - 2.0 * cross) + e2[None, :]
    tile, k = dist.shape
    mind = jnp.min(dist, axis=1)
    iota_k = lax.broadcasted_iota(jnp.int32, (tile, k), 1)
    amin = jnp.min(jnp.where(dist == mind[:, None], iota_k, k), axis=1)
    amin_ref[0, 0, :] = amin.astype(jnp.int32)

    @pl.when(pl.program_id(0) == 0)
    def _():
        lsum_ref[0, 0] = 0.0

    lsum_ref[0, 0] += jnp.sum(mind)


def _tc_argmin_loss(zf, emb, tile):
    n, d = zf.shape
    k = emb.shape[0]
    grid = n // tile
    return pl.pallas_call(
        _tc_body,
        grid=(grid,),
        in_specs=[
            pl.BlockSpec((tile, d), lambda i: (i, 0)),
            pl.BlockSpec((k, d), lambda i: (0, 0)),
        ],
        out_specs=[
            pl.BlockSpec((1, 1, tile), lambda i: (i, 0, 0)),
            pl.BlockSpec(memory_space=pltpu.SMEM),
        ],
        out_shape=[
            jax.ShapeDtypeStruct((grid, 1, tile), jnp.int32),
            jax.ShapeDtypeStruct((1, 1), jnp.float32),
        ],
    )(zf, emb)


def _sc_gather(emb, idx, n, d):
    info = plsc.get_sparse_core_info()
    nc, ns = info.num_cores, info.num_subcores
    nw = nc * ns
    b_per_w = n // nw
    ch = 128
    nch = b_per_w // ch
    mesh = plsc.VectorSubcoreMesh(core_axis_name="c", subcore_axis_name="s")

    @functools.partial(
        pl.kernel,
        mesh=mesh,
        out_type=jax.ShapeDtypeStruct((n, d), jnp.float32),
        scratch_types=[
            pltpu.VMEM((b_per_w,), jnp.int32),
            pltpu.VMEM((ch, d), jnp.float32),
            pltpu.VMEM((ch, d), jnp.float32),
            pltpu.SemaphoreType.DMA,
            pltpu.SemaphoreType.DMA,
        ],
    )
    def gather_kernel(emb_hbm, idx_hbm, out_hbm, idx_v, buf0, buf1, sem0, sem1):
        wid = lax.axis_index("s") * nc + lax.axis_index("c")
        base = wid * b_per_w
        pltpu.sync_copy(idx_hbm.at[pl.ds(base, b_per_w)], idx_v)
        bufs = (buf0, buf1)
        sems = (sem0, sem1)
        cps = [None, None]
        cps[0] = pltpu.async_copy(
            emb_hbm.at[idx_v.at[pl.ds(0, ch)]], bufs[0], sems[0])
        for c in range(nch):
            if c + 1 < nch:
                cps[(c + 1) % 2] = pltpu.async_copy(
                    emb_hbm.at[idx_v.at[pl.ds((c + 1) * ch, ch)]],
                    bufs[(c + 1) % 2], sems[(c + 1) % 2])
            cps[c % 2].wait()
            pltpu.sync_copy(bufs[c % 2], out_hbm.at[pl.ds(base + c * ch, ch)])

    return gather_kernel(emb, idx)


def kernel(z, emb):
    b, t, d = z.shape
    n = b * t
    zf = z.reshape(n, d)
    amin3, lsum = _tc_argmin_loss(zf, emb, tile=2048)
    amin_flat = amin3.reshape(n)
    z_q = _sc_gather(emb, amin_flat, n, d)
    loss = lsum[0, 0] * ((VQ_C + COMMIT_C) / (n * d))
    return z_q.reshape(b, t, d), amin_flat.reshape(b, t), loss
